# baseline (device time: 265119 ns/iter reference)
import jax
import jax.numpy as jnp
import numpy as np
from jax import lax
from jax.experimental import pallas as pl
from jax.experimental.pallas import tpu as pltpu

N_DEV = 4
SQ = 2048
D_MODEL = 1024
HQ_SH = 8
DH = 128
BQ = 128
NBQ = SQ // BQ
WIN = 3 * BQ
CH = SQ // N_DEV
SCALE = 0.08838834764831843


def _window_mask() -> np.ndarray:
    q_ids = np.arange(SQ).reshape(NBQ, BQ)
    k_ids = (np.arange(NBQ)[:, None] - 1) * BQ + np.arange(WIN)[None, :]
    valid = (k_ids >= 0) & (k_ids < SQ)
    win = np.abs(q_ids[:, :, None] - k_ids[:, None, :]) <= 128
    return win & valid[:, None, :]


def _ring_allreduce(partial):
    rows, cols = partial.shape

    def body(p_ref, out_ref, comm_ref, rs_send, rs_recv, ag_send, ag_recv):
        me = lax.axis_index("i")
        left = (me + N_DEV - 1) % N_DEV
        right = (me + 1) % N_DEV

        barrier = pltpu.get_barrier_semaphore()
        for nbr in (left, right):
            pl.semaphore_signal(
                barrier, inc=1, device_id=(nbr,),
                device_id_type=pl.DeviceIdType.MESH,
            )
        pl.semaphore_wait(barrier, 2)

        out_ref[...] = p_ref[...]

        for s in range(N_DEV - 1):
            send_c = (me - s) % N_DEV
            recv_c = (me - s - 1) % N_DEV
            rdma = pltpu.make_async_remote_copy(
                src_ref=out_ref.at[pl.ds(send_c * CH, CH), :],
                dst_ref=comm_ref.at[s],
                send_sem=rs_send.at[s],
                recv_sem=rs_recv.at[s],
                device_id=(right,),
                device_id_type=pl.DeviceIdType.MESH,
            )
            rdma.start()
            rdma.wait()
            out_ref[pl.ds(recv_c * CH, CH), :] += comm_ref[s]

        for s in range(N_DEV - 1):
            send_c = (me + 1 - s) % N_DEV
            rdma = pltpu.make_async_remote_copy(
                src_ref=out_ref.at[pl.ds(send_c * CH, CH), :],
                dst_ref=out_ref.at[pl.ds(send_c * CH, CH), :],
                send_sem=ag_send.at[s],
                recv_sem=ag_recv.at[s],
                device_id=(right,),
                device_id_type=pl.DeviceIdType.MESH,
            )
            rdma.start()
            rdma.wait()

    return pl.pallas_call(
        body,
        out_shape=jax.ShapeDtypeStruct((rows, cols), partial.dtype),
        in_specs=[pl.BlockSpec(memory_space=pltpu.VMEM)],
        out_specs=pl.BlockSpec(memory_space=pltpu.VMEM),
        scratch_shapes=[
            pltpu.VMEM((N_DEV - 1, CH, cols), partial.dtype),
            pltpu.SemaphoreType.DMA((N_DEV - 1,)),
            pltpu.SemaphoreType.DMA((N_DEV - 1,)),
            pltpu.SemaphoreType.DMA((N_DEV - 1,)),
            pltpu.SemaphoreType.DMA((N_DEV - 1,)),
        ],
        compiler_params=pltpu.CompilerParams(
            collective_id=0,
            vmem_limit_bytes=100 * 1024 * 1024,
        ),
    )(partial)


def kernel(x, Wq, K_ext, V_ext, Wo):
    me = lax.axis_index("i")

    x2 = x.reshape(SQ, D_MODEL)
    Q = (x2 @ Wq).reshape(SQ, HQ_SH, DH)
    K = lax.dynamic_slice_in_dim(
        K_ext.reshape(SQ, 32, DH), me * HQ_SH, HQ_SH, axis=1)
    V = lax.dynamic_slice_in_dim(
        V_ext.reshape(SQ, 32, DH), me * HQ_SH, HQ_SH, axis=1)

    Qb = Q.reshape(NBQ, BQ, HQ_SH, DH)
    Kp = jnp.pad(K, ((BQ, BQ), (0, 0), (0, 0)))
    Vp = jnp.pad(V, ((BQ, BQ), (0, 0), (0, 0)))

    def windows(a):
        shifts = [a[j * BQ: j * BQ + SQ].reshape(NBQ, BQ, HQ_SH, DH)
                  for j in range(3)]
        return jnp.concatenate(shifts, axis=1)

    Kw = windows(Kp)
    Vw = windows(Vp)

    scores = jnp.einsum("qihd,qjhd->qhij", Qb, Kw) * SCALE
    mask = jnp.asarray(_window_mask())
    scores = jnp.where(mask[:, None, :, :], scores, -1e9)
    scores_max = scores.max(axis=-1, keepdims=True)
    w = jnp.exp(scores - scores_max)
    w = w / w.sum(axis=-1, keepdims=True)
    ctx = jnp.einsum("qhij,qjhd->qihd", w, Vw)

    partial = ctx.reshape(SQ, HQ_SH * DH) @ Wo
    out = _ring_allreduce(partial)
    return out.reshape(1, SQ, D_MODEL)


# device time: 221631 ns/iter; 1.1962x vs baseline; 1.1962x over previous
import jax
import jax.numpy as jnp
import numpy as np
from jax import lax
from jax.experimental import pallas as pl
from jax.experimental.pallas import tpu as pltpu

N_DEV = 4
SQ = 2048
D_MODEL = 1024
HQ_SH = 8
DH = 128
BQ = 256
NBQ = SQ // BQ
KW = 512
CH = SQ // N_DEV
SCALE = 0.08838834764831843


def _attn_partial(x2, Wq, K_sh, V_sh, Wo):

    def body(x_ref, wq_ref, k_ref, v_ref, wo_ref, out_ref):
        i = pl.program_id(0)
        start = jnp.clip(BQ * i - 128, 0, SQ - KW)

        q = jnp.dot(x_ref[...], wq_ref[...],
                    preferred_element_type=jnp.float32)

        q_glob = BQ * i + lax.broadcasted_iota(jnp.int32, (BQ, KW), 0)
        k_glob = start + lax.broadcasted_iota(jnp.int32, (BQ, KW), 1)
        mask = jnp.abs(q_glob - k_glob) <= 128

        ctx_heads = []
        for h in range(HQ_SH):
            qh = q[:, h * DH:(h + 1) * DH]
            kh = k_ref[h, pl.ds(start, KW), :]
            vh = v_ref[h, pl.ds(start, KW), :]
            s = lax.dot_general(
                qh, kh, (((1,), (1,)), ((), ())),
                preferred_element_type=jnp.float32) * SCALE
            s = jnp.where(mask, s, -1e9)
            m = s.max(axis=1, keepdims=True)
            w = jnp.exp(s - m)
            w = w / w.sum(axis=1, keepdims=True)
            ctx_heads.append(jnp.dot(w, vh,
                                     preferred_element_type=jnp.float32))
        ctx = jnp.concatenate(ctx_heads, axis=1)
        out_ref[...] = jnp.dot(ctx, wo_ref[...],
                               preferred_element_type=jnp.float32)

    return pl.pallas_call(
        body,
        grid=(NBQ,),
        in_specs=[
            pl.BlockSpec((BQ, D_MODEL), lambda i: (i, 0)),
            pl.BlockSpec((D_MODEL, D_MODEL), lambda i: (0, 0)),
            pl.BlockSpec(memory_space=pltpu.VMEM),
            pl.BlockSpec(memory_space=pltpu.VMEM),
            pl.BlockSpec((D_MODEL, D_MODEL), lambda i: (0, 0)),
        ],
        out_specs=pl.BlockSpec((BQ, D_MODEL), lambda i: (i, 0)),
        out_shape=jax.ShapeDtypeStruct((SQ, D_MODEL), jnp.float32),
        compiler_params=pltpu.CompilerParams(
            vmem_limit_bytes=100 * 1024 * 1024,
        ),
    )(x2, Wq, K_sh, V_sh, Wo)


def _ring_allreduce(partial):
    rows, cols = partial.shape

    def body(p_ref, out_ref, comm_ref, rs_send, rs_recv, ag_send, ag_recv):
        me = lax.axis_index("i")
        left = (me + N_DEV - 1) % N_DEV
        right = (me + 1) % N_DEV

        barrier = pltpu.get_barrier_semaphore()
        for nbr in (left, right):
            pl.semaphore_signal(
                barrier, inc=1, device_id=(nbr,),
                device_id_type=pl.DeviceIdType.MESH,
            )
        pl.semaphore_wait(barrier, 2)

        out_ref[...] = p_ref[...]

        for s in range(N_DEV - 1):
            send_c = (me - s) % N_DEV
            recv_c = (me - s - 1) % N_DEV
            rdma = pltpu.make_async_remote_copy(
                src_ref=out_ref.at[pl.ds(send_c * CH, CH), :],
                dst_ref=comm_ref.at[s],
                send_sem=rs_send.at[s],
                recv_sem=rs_recv.at[s],
                device_id=(right,),
                device_id_type=pl.DeviceIdType.MESH,
            )
            rdma.start()
            rdma.wait()
            out_ref[pl.ds(recv_c * CH, CH), :] += comm_ref[s]

        for s in range(N_DEV - 1):
            send_c = (me + 1 - s) % N_DEV
            rdma = pltpu.make_async_remote_copy(
                src_ref=out_ref.at[pl.ds(send_c * CH, CH), :],
                dst_ref=out_ref.at[pl.ds(send_c * CH, CH), :],
                send_sem=ag_send.at[s],
                recv_sem=ag_recv.at[s],
                device_id=(right,),
                device_id_type=pl.DeviceIdType.MESH,
            )
            rdma.start()
            rdma.wait()

    return pl.pallas_call(
        body,
        out_shape=jax.ShapeDtypeStruct((rows, cols), partial.dtype),
        in_specs=[pl.BlockSpec(memory_space=pltpu.VMEM)],
        out_specs=pl.BlockSpec(memory_space=pltpu.VMEM),
        scratch_shapes=[
            pltpu.VMEM((N_DEV - 1, CH, cols), partial.dtype),
            pltpu.SemaphoreType.DMA((N_DEV - 1,)),
            pltpu.SemaphoreType.DMA((N_DEV - 1,)),
            pltpu.SemaphoreType.DMA((N_DEV - 1,)),
            pltpu.SemaphoreType.DMA((N_DEV - 1,)),
        ],
        compiler_params=pltpu.CompilerParams(
            collective_id=0,
            vmem_limit_bytes=100 * 1024 * 1024,
        ),
    )(partial)


def kernel(x, Wq, K_ext, V_ext, Wo):
    me = lax.axis_index("i")

    x2 = x.reshape(SQ, D_MODEL)
    K_sh = lax.dynamic_slice_in_dim(
        K_ext.reshape(SQ, 32, DH), me * HQ_SH, HQ_SH, axis=1).transpose(1, 0, 2)
    V_sh = lax.dynamic_slice_in_dim(
        V_ext.reshape(SQ, 32, DH), me * HQ_SH, HQ_SH, axis=1).transpose(1, 0, 2)

    partial = _attn_partial(x2, Wq, K_sh, V_sh, Wo)
    out = _ring_allreduce(partial)
    return out.reshape(1, SQ, D_MODEL)


# device time: 150887 ns/iter; 1.7571x vs baseline; 1.4689x over previous
import jax
import jax.numpy as jnp
import numpy as np
from jax import lax
from jax.experimental import pallas as pl
from jax.experimental.pallas import tpu as pltpu

N_DEV = 4
SQ = 2048
D_MODEL = 1024
HQ_SH = 8
DH = 128
BQ = 256
NBQ = SQ // BQ
KW = 512
CH = SQ // N_DEV
SCALE = 0.08838834764831843


def _attn_partial(x2, Wq, K_sh, V_sh, Wo):

    def body(x_ref, wq_ref, k_ref, v_ref, wo_ref, out_ref):
        i = pl.program_id(0)
        start = jnp.clip(BQ * i - 128, 0, SQ - KW)

        q = jnp.dot(x_ref[...], wq_ref[...],
                    preferred_element_type=jnp.float32)

        q_glob = BQ * i + lax.broadcasted_iota(jnp.int32, (BQ, KW), 0)
        k_glob = start + lax.broadcasted_iota(jnp.int32, (BQ, KW), 1)
        mask = jnp.abs(q_glob - k_glob) <= 128

        ctx_heads = []
        for h in range(HQ_SH):
            qh = q[:, h * DH:(h + 1) * DH]
            kh = k_ref[h, pl.ds(start, KW), :]
            vh = v_ref[h, pl.ds(start, KW), :]
            s = lax.dot_general(
                qh, kh, (((1,), (1,)), ((), ())),
                preferred_element_type=jnp.float32) * SCALE
            s = jnp.where(mask, s, -1e9)
            m = s.max(axis=1, keepdims=True)
            w = jnp.exp(s - m)
            w = w / w.sum(axis=1, keepdims=True)
            ctx_heads.append(jnp.dot(w, vh,
                                     preferred_element_type=jnp.float32))
        ctx = jnp.concatenate(ctx_heads, axis=1)
        out_ref[...] = jnp.dot(ctx, wo_ref[...],
                               preferred_element_type=jnp.float32)

    return pl.pallas_call(
        body,
        grid=(NBQ,),
        in_specs=[
            pl.BlockSpec((BQ, D_MODEL), lambda i: (i, 0)),
            pl.BlockSpec((D_MODEL, D_MODEL), lambda i: (0, 0)),
            pl.BlockSpec(memory_space=pltpu.VMEM),
            pl.BlockSpec(memory_space=pltpu.VMEM),
            pl.BlockSpec((D_MODEL, D_MODEL), lambda i: (0, 0)),
        ],
        out_specs=pl.BlockSpec((BQ, D_MODEL), lambda i: (i, 0)),
        out_shape=jax.ShapeDtypeStruct((SQ, D_MODEL), jnp.float32),
        compiler_params=pltpu.CompilerParams(
            vmem_limit_bytes=100 * 1024 * 1024,
        ),
    )(x2, Wq, K_sh, V_sh, Wo)


def _butterfly_allreduce(partial):
    rows, cols = partial.shape
    half_r = rows // 2
    qr = rows // 4
    hc = cols // 2

    def body(p_ref, out_ref, c1a, c1b, c2a, c2b,
             sa_send, sa_recv, sb_send, sb_recv):
        me = lax.axis_index("i")
        py = me ^ 1
        px = 3 - me

        barrier = pltpu.get_barrier_semaphore()
        for nbr in (py, px):
            pl.semaphore_signal(
                barrier, inc=1, device_id=(nbr,),
                device_id_type=pl.DeviceIdType.MESH,
            )
        pl.semaphore_wait(barrier, 2)

        out_ref[...] = p_ref[...]

        fa = (me ^ (me >> 1)) & 1
        ga = (me >> 1) & 1
        fb = (me >> 1) & 1
        gb = me & 1

        def xchg(src_rows, n_rows, col0, dst, dst_rows, sems, phase, dev):
            dst_ref = (dst if dst_rows is None
                       else dst.at[pl.ds(dst_rows, n_rows), pl.ds(col0, hc)])
            return pltpu.make_async_remote_copy(
                src_ref=out_ref.at[pl.ds(src_rows, n_rows), pl.ds(col0, hc)],
                dst_ref=dst_ref,
                send_sem=sems[0].at[phase],
                recv_sem=sems[1].at[phase],
                device_id=(dev,),
                device_id_type=pl.DeviceIdType.MESH,
            )

        sa = (sa_send, sa_recv)
        sb = (sb_send, sb_recv)

        a = xchg((1 - fa) * half_r, half_r, 0, c1a, None, sa, 0, py)
        b = xchg((1 - fb) * half_r, half_r, hc, c1b, None, sb, 0, px)
        a.start(); b.start(); a.wait(); b.wait()
        out_ref[pl.ds(fa * half_r, half_r), 0:hc] += c1a[...]
        out_ref[pl.ds(fb * half_r, half_r), hc:cols] += c1b[...]

        qka = 2 * fa + ga
        qsa = 2 * fa + (1 - ga)
        qkb = 2 * fb + gb
        qsb = 2 * fb + (1 - gb)

        a = xchg(qsa * qr, qr, 0, c2a, None, sa, 1, px)
        b = xchg(qsb * qr, qr, hc, c2b, None, sb, 1, py)
        a.start(); b.start(); a.wait(); b.wait()
        out_ref[pl.ds(qka * qr, qr), 0:hc] += c2a[...]
        out_ref[pl.ds(qkb * qr, qr), hc:cols] += c2b[...]

        a = xchg(qka * qr, qr, 0, out_ref, qka * qr, sa, 2, px)
        b = xchg(qkb * qr, qr, hc, out_ref, qkb * qr, sb, 2, py)
        a.start(); b.start(); a.wait(); b.wait()

        a = xchg(fa * half_r, half_r, 0, out_ref, fa * half_r, sa, 3, py)
        b = xchg(fb * half_r, half_r, hc, out_ref, fb * half_r, sb, 3, px)
        a.start(); b.start(); a.wait(); b.wait()

    return pl.pallas_call(
        body,
        out_shape=jax.ShapeDtypeStruct((rows, cols), partial.dtype),
        in_specs=[pl.BlockSpec(memory_space=pltpu.VMEM)],
        out_specs=pl.BlockSpec(memory_space=pltpu.VMEM),
        scratch_shapes=[
            pltpu.VMEM((half_r, hc), partial.dtype),
            pltpu.VMEM((half_r, hc), partial.dtype),
            pltpu.VMEM((qr, hc), partial.dtype),
            pltpu.VMEM((qr, hc), partial.dtype),
            pltpu.SemaphoreType.DMA((4,)),
            pltpu.SemaphoreType.DMA((4,)),
            pltpu.SemaphoreType.DMA((4,)),
            pltpu.SemaphoreType.DMA((4,)),
        ],
        compiler_params=pltpu.CompilerParams(
            collective_id=0,
            vmem_limit_bytes=100 * 1024 * 1024,
        ),
    )(partial)


def kernel(x, Wq, K_ext, V_ext, Wo):
    me = lax.axis_index("i")

    x2 = x.reshape(SQ, D_MODEL)
    K_sh = lax.dynamic_slice_in_dim(
        K_ext.reshape(SQ, 32, DH), me * HQ_SH, HQ_SH, axis=1).transpose(1, 0, 2)
    V_sh = lax.dynamic_slice_in_dim(
        V_ext.reshape(SQ, 32, DH), me * HQ_SH, HQ_SH, axis=1).transpose(1, 0, 2)

    partial = _attn_partial(x2, Wq, K_sh, V_sh, Wo)
    out = _butterfly_allreduce(partial)
    return out.reshape(1, SQ, D_MODEL)


# device time: 150533 ns/iter; 1.7612x vs baseline; 1.0024x over previous
import jax
import jax.numpy as jnp
import numpy as np
from jax import lax
from jax.experimental import pallas as pl
from jax.experimental.pallas import tpu as pltpu

N_DEV = 4
SQ = 2048
D_MODEL = 1024
HQ_SH = 8
DH = 128
BQ = 256
NBQ = SQ // BQ
KW = 512
CH = SQ // N_DEV
SCALE = 0.08838834764831843


def _attn_partial(x2, Wq, K_sh, V_sh, Wo):

    def body(x_ref, wq_ref, k_ref, v_ref, wo_ref, out_ref):
        i = pl.program_id(0)
        start = jnp.clip(BQ * i - 128, 0, SQ - KW)

        q = jnp.dot(x_ref[...], wq_ref[...],
                    preferred_element_type=jnp.float32)

        q_glob = BQ * i + lax.broadcasted_iota(jnp.int32, (BQ, KW), 0)
        k_glob = start + lax.broadcasted_iota(jnp.int32, (BQ, KW), 1)
        mask = jnp.abs(q_glob - k_glob) <= 128

        ctx_heads = []
        for h in range(HQ_SH):
            qh = q[:, h * DH:(h + 1) * DH]
            kh = k_ref[h, pl.ds(start, KW), :]
            vh = v_ref[h, pl.ds(start, KW), :]
            s = lax.dot_general(
                qh, kh, (((1,), (1,)), ((), ())),
                preferred_element_type=jnp.float32) * SCALE
            s = jnp.where(mask, s, -1e9)
            m = s.max(axis=1, keepdims=True)
            w = jnp.exp(s - m)
            w = w / w.sum(axis=1, keepdims=True)
            ctx_heads.append(jnp.dot(w, vh,
                                     preferred_element_type=jnp.float32))
        ctx = jnp.concatenate(ctx_heads, axis=1)
        out_ref[...] = jnp.dot(ctx, wo_ref[...],
                               preferred_element_type=jnp.float32)

    return pl.pallas_call(
        body,
        grid=(NBQ,),
        in_specs=[
            pl.BlockSpec((BQ, D_MODEL), lambda i: (i, 0)),
            pl.BlockSpec((D_MODEL, D_MODEL), lambda i: (0, 0)),
            pl.BlockSpec(memory_space=pltpu.VMEM),
            pl.BlockSpec(memory_space=pltpu.VMEM),
            pl.BlockSpec((D_MODEL, D_MODEL), lambda i: (0, 0)),
        ],
        out_specs=pl.BlockSpec((BQ, D_MODEL), lambda i: (i, 0)),
        out_shape=jax.ShapeDtypeStruct((SQ, D_MODEL), jnp.float32),
        compiler_params=pltpu.CompilerParams(
            vmem_limit_bytes=100 * 1024 * 1024,
        ),
    )(x2, Wq, K_sh, V_sh, Wo)


def _butterfly_allreduce(partial):
    rows, cols = partial.shape
    hr = rows // 4
    qr = rows // 8
    base_b = rows // 2

    def body(p_ref, out_ref, c1a, c1b, c2a, c2b,
             sa_send, sa_recv, sb_send, sb_recv):
        me = lax.axis_index("i")
        py = me ^ 1
        px = 3 - me

        barrier = pltpu.get_barrier_semaphore()
        for nbr in (py, px):
            pl.semaphore_signal(
                barrier, inc=1, device_id=(nbr,),
                device_id_type=pl.DeviceIdType.MESH,
            )
        pl.semaphore_wait(barrier, 2)

        out_ref[...] = p_ref[...]

        fa = (me ^ (me >> 1)) & 1
        ga = (me >> 1) & 1
        fb = (me >> 1) & 1
        gb = me & 1

        def xchg(src_rows, n_rows, dst, dst_rows, sems, phase, dev):
            dst_ref = (dst if dst_rows is None
                       else dst.at[pl.ds(dst_rows, n_rows), :])
            return pltpu.make_async_remote_copy(
                src_ref=out_ref.at[pl.ds(src_rows, n_rows), :],
                dst_ref=dst_ref,
                send_sem=sems[0].at[phase],
                recv_sem=sems[1].at[phase],
                device_id=(dev,),
                device_id_type=pl.DeviceIdType.MESH,
            )

        sa = (sa_send, sa_recv)
        sb = (sb_send, sb_recv)

        a = xchg((1 - fa) * hr, hr, c1a, None, sa, 0, py)
        b = xchg(base_b + (1 - fb) * hr, hr, c1b, None, sb, 0, px)
        a.start(); b.start(); a.wait(); b.wait()
        out_ref[pl.ds(fa * hr, hr), :] += c1a[...]
        out_ref[pl.ds(base_b + fb * hr, hr), :] += c1b[...]

        qka = 2 * fa + ga
        qsa = 2 * fa + (1 - ga)
        qkb = 2 * fb + gb
        qsb = 2 * fb + (1 - gb)

        a = xchg(qsa * qr, qr, c2a, None, sa, 1, px)
        b = xchg(base_b + qsb * qr, qr, c2b, None, sb, 1, py)
        a.start(); b.start(); a.wait(); b.wait()
        out_ref[pl.ds(qka * qr, qr), :] += c2a[...]
        out_ref[pl.ds(base_b + qkb * qr, qr), :] += c2b[...]

        a = xchg(qka * qr, qr, out_ref, qka * qr, sa, 2, px)
        b = xchg(base_b + qkb * qr, qr, out_ref, base_b + qkb * qr, sb, 2, py)
        a.start(); b.start(); a.wait(); b.wait()

        a = xchg(fa * hr, hr, out_ref, fa * hr, sa, 3, py)
        b = xchg(base_b + fb * hr, hr, out_ref, base_b + fb * hr, sb, 3, px)
        a.start(); b.start(); a.wait(); b.wait()

    return pl.pallas_call(
        body,
        out_shape=jax.ShapeDtypeStruct((rows, cols), partial.dtype),
        in_specs=[pl.BlockSpec(memory_space=pltpu.VMEM)],
        out_specs=pl.BlockSpec(memory_space=pltpu.VMEM),
        scratch_shapes=[
            pltpu.VMEM((hr, cols), partial.dtype),
            pltpu.VMEM((hr, cols), partial.dtype),
            pltpu.VMEM((qr, cols), partial.dtype),
            pltpu.VMEM((qr, cols), partial.dtype),
            pltpu.SemaphoreType.DMA((4,)),
            pltpu.SemaphoreType.DMA((4,)),
            pltpu.SemaphoreType.DMA((4,)),
            pltpu.SemaphoreType.DMA((4,)),
        ],
        compiler_params=pltpu.CompilerParams(
            collective_id=0,
            vmem_limit_bytes=100 * 1024 * 1024,
        ),
    )(partial)


def kernel(x, Wq, K_ext, V_ext, Wo):
    me = lax.axis_index("i")

    x2 = x.reshape(SQ, D_MODEL)
    K_sh = lax.dynamic_slice_in_dim(
        K_ext.reshape(SQ, 32, DH), me * HQ_SH, HQ_SH, axis=1).transpose(1, 0, 2)
    V_sh = lax.dynamic_slice_in_dim(
        V_ext.reshape(SQ, 32, DH), me * HQ_SH, HQ_SH, axis=1).transpose(1, 0, 2)

    partial = _attn_partial(x2, Wq, K_sh, V_sh, Wo)
    out = _butterfly_allreduce(partial)
    return out.reshape(1, SQ, D_MODEL)


# device time: 132505 ns/iter; 2.0008x vs baseline; 1.1361x over previous
import jax
import jax.numpy as jnp
import numpy as np
from jax import lax
from jax.experimental import pallas as pl
from jax.experimental.pallas import tpu as pltpu

N_DEV = 4
SQ = 2048
D_MODEL = 1024
HQ_SH = 8
DH = 128
BQ = 256
NBQ = SQ // BQ
KW = 512
HR = SQ // 4
QR = SQ // 8
BASE_B = SQ // 2
SCALE = 0.08838834764831843


def _fused(x2, Wq, K_sh, V_sh, Wo):
    def body(x_ref, wq_ref, k_ref, v_ref, wo_ref, out_ref,
             c1a, c1b, c2a, c2b, sa_send, sa_recv, sb_send, sb_recv):
        me = lax.axis_index("i")
        py = me ^ 1
        px = 3 - me

        barrier = pltpu.get_barrier_semaphore()
        for nbr in (py, px):
            pl.semaphore_signal(
                barrier, inc=1, device_id=(nbr,),
                device_id_type=pl.DeviceIdType.MESH,
            )
        pl.semaphore_wait(barrier, 2)

        def compute_block(row0):
            start = jnp.clip(row0 - 128, 0, SQ - KW)
            q = jnp.dot(x_ref[pl.ds(row0, BQ), :], wq_ref[...],
                        preferred_element_type=jnp.float32)
            q_glob = row0 + lax.broadcasted_iota(jnp.int32, (BQ, KW), 0)
            k_glob = start + lax.broadcasted_iota(jnp.int32, (BQ, KW), 1)
            mask = jnp.abs(q_glob - k_glob) <= 128
            ctx_heads = []
            for h in range(HQ_SH):
                qh = q[:, h * DH:(h + 1) * DH]
                kh = k_ref[h, pl.ds(start, KW), :]
                vh = v_ref[h, pl.ds(start, KW), :]
                s = lax.dot_general(
                    qh, kh, (((1,), (1,)), ((), ())),
                    preferred_element_type=jnp.float32) * SCALE
                s = jnp.where(mask, s, -1e9)
                m = s.max(axis=1, keepdims=True)
                w = jnp.exp(s - m)
                w = w / w.sum(axis=1, keepdims=True)
                ctx_heads.append(jnp.dot(w, vh,
                                         preferred_element_type=jnp.float32))
            ctx = jnp.concatenate(ctx_heads, axis=1)
            out_ref[pl.ds(row0, BQ), :] = jnp.dot(
                ctx, wo_ref[...], preferred_element_type=jnp.float32)

        fa = (me ^ (me >> 1)) & 1
        ga = (me >> 1) & 1
        fb = (me >> 1) & 1
        gb = me & 1

        def xchg(src_rows, n_rows, dst, dst_rows, sems, phase, dev):
            dst_ref = (dst if dst_rows is None
                       else dst.at[pl.ds(dst_rows, n_rows), :])
            return pltpu.make_async_remote_copy(
                src_ref=out_ref.at[pl.ds(src_rows, n_rows), :],
                dst_ref=dst_ref,
                send_sem=sems[0].at[phase],
                recv_sem=sems[1].at[phase],
                device_id=(dev,),
                device_id_type=pl.DeviceIdType.MESH,
            )

        sa = (sa_send, sa_recv)
        sb = (sb_send, sb_recv)

        a_send_r = (1 - fa) * HR
        b_send_r = BASE_B + (1 - fb) * HR
        a_keep_r = fa * HR
        b_keep_r = BASE_B + fb * HR
        for j in range(2):
            compute_block(a_send_r + j * BQ)
        for j in range(2):
            compute_block(b_send_r + j * BQ)
        a = xchg(a_send_r, HR, c1a, None, sa, 0, py)
        b = xchg(b_send_r, HR, c1b, None, sb, 0, px)
        a.start()
        b.start()
        for j in range(2):
            compute_block(a_keep_r + j * BQ)
        for j in range(2):
            compute_block(b_keep_r + j * BQ)
        a.wait()
        b.wait()
        out_ref[pl.ds(a_keep_r, HR), :] += c1a[...]
        out_ref[pl.ds(b_keep_r, HR), :] += c1b[...]

        qka = 2 * fa + ga
        qsa = 2 * fa + (1 - ga)
        qkb = 2 * fb + gb
        qsb = 2 * fb + (1 - gb)

        a = xchg(qsa * QR, QR, c2a, None, sa, 1, px)
        b = xchg(BASE_B + qsb * QR, QR, c2b, None, sb, 1, py)
        a.start(); b.start(); a.wait(); b.wait()
        out_ref[pl.ds(qka * QR, QR), :] += c2a[...]
        out_ref[pl.ds(BASE_B + qkb * QR, QR), :] += c2b[...]

        a = xchg(qka * QR, QR, out_ref, qka * QR, sa, 2, px)
        b = xchg(BASE_B + qkb * QR, QR, out_ref, BASE_B + qkb * QR, sb, 2, py)
        a.start(); b.start(); a.wait(); b.wait()

        a = xchg(fa * HR, HR, out_ref, fa * HR, sa, 3, py)
        b = xchg(BASE_B + fb * HR, HR, out_ref, BASE_B + fb * HR, sb, 3, px)
        a.start(); b.start(); a.wait(); b.wait()

    return pl.pallas_call(
        body,
        out_shape=jax.ShapeDtypeStruct((SQ, D_MODEL), jnp.float32),
        in_specs=[pl.BlockSpec(memory_space=pltpu.VMEM)] * 5,
        out_specs=pl.BlockSpec(memory_space=pltpu.VMEM),
        scratch_shapes=[
            pltpu.VMEM((HR, D_MODEL), jnp.float32),
            pltpu.VMEM((HR, D_MODEL), jnp.float32),
            pltpu.VMEM((QR, D_MODEL), jnp.float32),
            pltpu.VMEM((QR, D_MODEL), jnp.float32),
            pltpu.SemaphoreType.DMA((4,)),
            pltpu.SemaphoreType.DMA((4,)),
            pltpu.SemaphoreType.DMA((4,)),
            pltpu.SemaphoreType.DMA((4,)),
        ],
        compiler_params=pltpu.CompilerParams(
            collective_id=0,
            vmem_limit_bytes=100 * 1024 * 1024,
        ),
    )(x2, Wq, K_sh, V_sh, Wo)


def kernel(x, Wq, K_ext, V_ext, Wo):
    me = lax.axis_index("i")

    x2 = x.reshape(SQ, D_MODEL)
    K_sh = lax.dynamic_slice_in_dim(
        K_ext.reshape(SQ, 32, DH), me * HQ_SH, HQ_SH, axis=1).transpose(1, 0, 2)
    V_sh = lax.dynamic_slice_in_dim(
        V_ext.reshape(SQ, 32, DH), me * HQ_SH, HQ_SH, axis=1).transpose(1, 0, 2)

    out = _fused(x2, Wq, K_sh, V_sh, Wo)
    return out.reshape(1, SQ, D_MODEL)
